# baseline (speedup 1.0000x reference)
import jax
import jax.numpy as jnp
from jax.experimental import pallas as pl
from jax.experimental.pallas import tpu as pltpu

_B_TILE = 2048
_ENSEMBLE = 8


def _fused_kernel(x_ref, w_ref, h_ref, out_ref):
    x = x_ref[...]
    h = h_ref[...]
    best = None
    best_crc = None
    for i in range(_ENSEMBLE):
        d = jax.nn.sigmoid(
            jnp.dot(x, w_ref[i], preferred_element_type=jnp.float32))
        hm = jax.lax.dot_general(
            d, h, (((1,), (1,)), ((), ())),
            preferred_element_type=jnp.float32)
        m2 = hm - 2.0 * jnp.floor(hm * 0.5)
        crc = jnp.sum(m2, axis=1, keepdims=True)
        if i == 0:
            best, best_crc = d, crc
        else:
            take = crc < best_crc
            best = jnp.where(take, d, best)
            best_crc = jnp.where(take, crc, best_crc)
    out_ref[...] = best


def kernel(x, W, code_h_outer):
    batch, in_len = x.shape
    ens, _, det = W.shape
    h_rows = code_h_outer.shape[0]
    return pl.pallas_call(
        _fused_kernel,
        grid=(batch // _B_TILE,),
        in_specs=[
            pl.BlockSpec((_B_TILE, in_len), lambda i: (i, 0)),
            pl.BlockSpec((ens, in_len, det), lambda i: (0, 0, 0)),
            pl.BlockSpec((h_rows, det), lambda i: (0, 0)),
        ],
        out_specs=pl.BlockSpec((_B_TILE, det), lambda i: (i, 0)),
        out_shape=jax.ShapeDtypeStruct((batch, det), jnp.float32),
        compiler_params=pltpu.CompilerParams(
            dimension_semantics=("parallel",)),
    )(x, W, code_h_outer)


# speedup vs baseline: 1.1573x; 1.1573x over previous
import jax
import jax.numpy as jnp
from jax.experimental import pallas as pl
from jax.experimental.pallas import tpu as pltpu

_B_TILE = 2048
_ENSEMBLE = 8


def _fused_kernel(x_ref, w_ref, h_ref, out_ref):
    x = x_ref[...]
    h = h_ref[...]
    det = h.shape[1]
    d_all = jax.nn.sigmoid(
        jnp.dot(x, w_ref[...], preferred_element_type=jnp.float32))
    best = None
    best_crc = None
    for i in range(_ENSEMBLE):
        d = d_all[:, i * det:(i + 1) * det]
        hm = jax.lax.dot_general(
            d, h, (((1,), (1,)), ((), ())),
            preferred_element_type=jnp.float32)
        m2 = hm - jnp.floor(hm)
        crc = jnp.sum(m2, axis=1, keepdims=True)
        if i == 0:
            best, best_crc = d, crc
        else:
            take = crc < best_crc
            best = jnp.where(take, d, best)
            best_crc = jnp.where(take, crc, best_crc)
    out_ref[...] = best


def kernel(x, W, code_h_outer):
    batch, in_len = x.shape
    ens, _, det = W.shape
    h_rows = code_h_outer.shape[0]
    w_flat = W.transpose(1, 0, 2).reshape(in_len, ens * det)
    return pl.pallas_call(
        _fused_kernel,
        grid=(batch // _B_TILE,),
        in_specs=[
            pl.BlockSpec((_B_TILE, in_len), lambda i: (i, 0)),
            pl.BlockSpec((in_len, ens * det), lambda i: (0, 0)),
            pl.BlockSpec((h_rows, det), lambda i: (0, 0)),
        ],
        out_specs=pl.BlockSpec((_B_TILE, det), lambda i: (i, 0)),
        out_shape=jax.ShapeDtypeStruct((batch, det), jnp.float32),
        compiler_params=pltpu.CompilerParams(
            dimension_semantics=("parallel",)),
    )(x, w_flat, code_h_outer * 0.5)
